# baseline (speedup 1.0000x reference)
import functools

import jax
import jax.numpy as jnp
from jax import lax
from jax.experimental import pallas as pl
from jax.experimental.pallas import tpu as pltpu
from jax.experimental.pallas import tpu_sc as plsc

ITEM_LEN = 1000000
EMBED_DIM = 64
BATCH = 16384

_info = plsc.get_sparse_core_info()
NUM_CORES = _info.num_cores
NUM_SUBCORES = _info.num_subcores
LANES = _info.num_lanes
NUM_WORKERS = NUM_CORES * NUM_SUBCORES
B_PER_W = BATCH // NUM_WORKERS
HALF = B_PER_W // 2
STEPS = HALF // LANES
NSEM = 8


def _sc_kernel(ti_hbm, cj_hbm, table_hbm, out_hbm,
               ti_v, cj_v, tbuf, cbuf, out_v, sem_t, sem_c):
    wid = lax.axis_index("s") * NUM_CORES + lax.axis_index("c")
    base = wid * B_PER_W

    pltpu.sync_copy(ti_hbm.at[pl.ds(base, B_PER_W)], ti_v)
    pltpu.sync_copy(cj_hbm.at[pl.ds(base, B_PER_W)], cj_v)

    iota = lax.iota(jnp.int32, LANES)

    for h in range(2):
        def fire(j, carry):
            tvec = ti_v[pl.ds(h * HALF + j * LANES, LANES)]
            cvec = cj_v[pl.ds(h * HALF + j * LANES, LANES)]
            for l in range(LANES):
                rt = tvec[l]
                rc = cvec[l]
                slot = j * LANES + l
                pltpu.async_copy(
                    table_hbm.at[pl.ds(rt, 1), :],
                    tbuf.at[pl.ds(slot, 1), pl.ds(0, EMBED_DIM)],
                    sem_t.at[l % NSEM])
                pltpu.async_copy(
                    table_hbm.at[pl.ds(rc, 1), :],
                    cbuf.at[pl.ds(slot, 1), pl.ds(0, EMBED_DIM)],
                    sem_c.at[l % NSEM])
            return carry

        lax.fori_loop(0, STEPS, fire, 0)

        per_sem = HALF // NSEM
        for k in range(NSEM):
            pltpu.make_async_copy(
                table_hbm.at[pl.ds(0, per_sem), :],
                tbuf.at[pl.ds(0, per_sem), :], sem_t.at[k]).wait()
            pltpu.make_async_copy(
                table_hbm.at[pl.ds(0, per_sem), :],
                cbuf.at[pl.ds(0, per_sem), :], sem_c.at[k]).wait()

        def compute(g, carry):
            rv = g * LANES + iota
            acc = None
            for d in range(EMBED_DIM):
                dvec = jnp.full((LANES,), d, dtype=jnp.int32)
                tv = plsc.load_gather(tbuf, [rv, dvec])
                cv = plsc.load_gather(cbuf, [rv, dvec])
                prod = tv * cv
                acc = prod if acc is None else acc + prod
            y = 1.0 / (1.0 + jnp.exp(-acc))
            out_v[pl.ds(h * HALF + g * LANES, LANES)] = y
            return carry

        lax.fori_loop(0, STEPS, compute, 0)

    pltpu.sync_copy(out_v, out_hbm.at[pl.ds(base, B_PER_W)])


@jax.jit
def kernel(target_i, context_j, embedding_table):
    mesh = plsc.VectorSubcoreMesh(core_axis_name="c", subcore_axis_name="s")
    f = functools.partial(
        pl.kernel,
        out_type=jax.ShapeDtypeStruct((BATCH,), jnp.float32),
        mesh=mesh,
        compiler_params=pltpu.CompilerParams(
            needs_layout_passes=False, use_tc_tiling_on_sc=True),
        scratch_types=[
            pltpu.VMEM((B_PER_W,), jnp.int32),
            pltpu.VMEM((B_PER_W,), jnp.int32),
            pltpu.VMEM((HALF, EMBED_DIM), jnp.float32),
            pltpu.VMEM((HALF, EMBED_DIM), jnp.float32),
            pltpu.VMEM((B_PER_W,), jnp.float32),
            pltpu.SemaphoreType.DMA((NSEM,)),
            pltpu.SemaphoreType.DMA((NSEM,)),
        ],
    )(_sc_kernel)
    return f(target_i.astype(jnp.int32), context_j.astype(jnp.int32),
             embedding_table)


# speedup vs baseline: 1.1522x; 1.1522x over previous
import functools

import jax
import jax.numpy as jnp
from jax import lax
from jax.experimental import pallas as pl
from jax.experimental.pallas import tpu as pltpu
from jax.experimental.pallas import tpu_sc as plsc

ITEM_LEN = 1000000
EMBED_DIM = 64
BATCH = 16384

_info = plsc.get_sparse_core_info()
NUM_CORES = _info.num_cores
NUM_SUBCORES = _info.num_subcores
LANES = _info.num_lanes
NUM_WORKERS = NUM_CORES * NUM_SUBCORES
B_PER_W = BATCH // NUM_WORKERS
QUARTER = B_PER_W // 4
STEPS = QUARTER // LANES


def _sc_kernel(ti_hbm, cj_hbm, table_hbm, out_hbm,
               ti_v, cj_v, tbufs, cbufs, out_v, sem_t, sem_c):
    wid = lax.axis_index("s") * NUM_CORES + lax.axis_index("c")
    base = wid * B_PER_W

    pltpu.sync_copy(ti_hbm.at[pl.ds(base, B_PER_W)], ti_v)
    pltpu.sync_copy(cj_hbm.at[pl.ds(base, B_PER_W)], cj_v)

    iota = lax.iota(jnp.int32, LANES)

    def fire(q):
        tbuf, cbuf = tbufs[q % 2], cbufs[q % 2]
        st, sc = sem_t.at[q % 2], sem_c.at[q % 2]

        def body(j, carry):
            tvec = ti_v[pl.ds(q * QUARTER + j * LANES, LANES)]
            cvec = cj_v[pl.ds(q * QUARTER + j * LANES, LANES)]
            for l in range(LANES):
                rt = tvec[l]
                rc = cvec[l]
                slot = j * LANES + l
                pltpu.async_copy(
                    table_hbm.at[pl.ds(rt, 1), :],
                    tbuf.at[pl.ds(slot, 1), pl.ds(0, EMBED_DIM)], st)
                pltpu.async_copy(
                    table_hbm.at[pl.ds(rc, 1), :],
                    cbuf.at[pl.ds(slot, 1), pl.ds(0, EMBED_DIM)], sc)
            return carry

        lax.fori_loop(0, STEPS, body, 0)

    def drain_and_compute(q):
        tbuf, cbuf = tbufs[q % 2], cbufs[q % 2]
        pltpu.make_async_copy(
            table_hbm.at[pl.ds(0, QUARTER), :], tbuf, sem_t.at[q % 2]).wait()
        pltpu.make_async_copy(
            table_hbm.at[pl.ds(0, QUARTER), :], cbuf, sem_c.at[q % 2]).wait()

        def body(g, carry):
            rv = g * LANES + iota
            acc = None
            for d in range(EMBED_DIM):
                dvec = jnp.full((LANES,), d, dtype=jnp.int32)
                tv = plsc.load_gather(tbuf, [rv, dvec])
                cv = plsc.load_gather(cbuf, [rv, dvec])
                prod = tv * cv
                acc = prod if acc is None else acc + prod
            y = 1.0 / (1.0 + jnp.exp(-acc))
            out_v[pl.ds(q * QUARTER + g * LANES, LANES)] = y
            return carry

        lax.fori_loop(0, STEPS, body, 0)

    fire(0)
    for q in range(4):
        if q + 1 < 4:
            fire(q + 1)
        drain_and_compute(q)

    pltpu.sync_copy(out_v, out_hbm.at[pl.ds(base, B_PER_W)])


@jax.jit
def kernel(target_i, context_j, embedding_table):
    mesh = plsc.VectorSubcoreMesh(core_axis_name="c", subcore_axis_name="s")
    f = functools.partial(
        pl.kernel,
        out_type=jax.ShapeDtypeStruct((BATCH,), jnp.float32),
        mesh=mesh,
        compiler_params=pltpu.CompilerParams(
            needs_layout_passes=False, use_tc_tiling_on_sc=True),
        scratch_types=[
            pltpu.VMEM((B_PER_W,), jnp.int32),
            pltpu.VMEM((B_PER_W,), jnp.int32),
            [pltpu.VMEM((QUARTER, EMBED_DIM), jnp.float32) for _ in range(2)],
            [pltpu.VMEM((QUARTER, EMBED_DIM), jnp.float32) for _ in range(2)],
            pltpu.VMEM((B_PER_W,), jnp.float32),
            pltpu.SemaphoreType.DMA((2,)),
            pltpu.SemaphoreType.DMA((2,)),
        ],
    )(_sc_kernel)
    return f(target_i.astype(jnp.int32), context_j.astype(jnp.int32),
             embedding_table)
